# baseline (device time: 24967 ns/iter reference)
import jax
import jax.numpy as jnp
from jax import lax
from jax.experimental import pallas as pl
from jax.experimental.pallas import tpu as pltpu

N_DEV = 8
_GELU_C = 0.7978845608028654


def _gelu_bf16(y):
    y = 0.5 * y * (1.0 + jnp.tanh(_GELU_C * (y + 0.044715 * y * y * y)))
    return y.astype(jnp.bfloat16)


def kernel(x, w_mat):
    m_per, k = x.shape
    _, n = w_mat.shape
    blk = n // N_DEV

    def body(x_ref, w_hbm, out_ref, w_ref, xb_ref, y_ref,
             copy_sems, send_sems, recv_sems):
        my_i = lax.axis_index("i")

        barrier_sem = pltpu.get_barrier_semaphore()
        for j in range(1, N_DEV):
            pl.semaphore_signal(
                barrier_sem, inc=1,
                device_id=((my_i + j) % N_DEV,),
                device_id_type=pl.DeviceIdType.MESH,
            )
        pl.semaphore_wait(barrier_sem, N_DEV - 1)

        copies = []
        for j in range(1, N_DEV + 1):
            p = (my_i + j) % N_DEV
            cp = pltpu.make_async_copy(
                w_hbm.at[:, pl.ds(p * blk, blk)],
                w_ref.at[j % N_DEV],
                copy_sems.at[j % N_DEV],
            )
            cp.start()
            copies.append(cp)

        xb_ref[...] = x_ref[...].astype(jnp.bfloat16)

        sends = []
        for j in range(1, N_DEV):
            p = (my_i + j) % N_DEV
            copies[j - 1].wait()
            wb = w_ref[j, :, :].astype(jnp.bfloat16)
            y_ref[j, :, :] = _gelu_bf16(
                jnp.dot(xb_ref[...], wb, preferred_element_type=jnp.float32)
            )
            rdma = pltpu.make_async_remote_copy(
                src_ref=y_ref.at[j],
                dst_ref=out_ref.at[pl.ds(my_i * m_per, m_per), :],
                send_sem=send_sems.at[j],
                recv_sem=recv_sems.at[j],
                device_id=(p,),
                device_id_type=pl.DeviceIdType.MESH,
            )
            rdma.start()
            sends.append(rdma)

        copies[N_DEV - 1].wait()
        wb = w_ref[0, :, :].astype(jnp.bfloat16)
        out_ref[pl.ds(my_i * m_per, m_per), :] = _gelu_bf16(
            jnp.dot(xb_ref[...], wb, preferred_element_type=jnp.float32)
        )

        for j in range(1, N_DEV):
            s = (my_i - j) % N_DEV
            recv = pltpu.make_async_remote_copy(
                src_ref=y_ref.at[j],
                dst_ref=out_ref.at[pl.ds(s * m_per, m_per), :],
                send_sem=send_sems.at[j],
                recv_sem=recv_sems.at[j],
                device_id=(s,),
                device_id_type=pl.DeviceIdType.MESH,
            )
            recv.wait_recv()

        for rdma in sends:
            rdma.wait_send()

    out_shape = jax.ShapeDtypeStruct((N_DEV * m_per, blk), jnp.bfloat16)
    return pl.pallas_call(
        body,
        out_shape=out_shape,
        in_specs=[
            pl.BlockSpec(memory_space=pltpu.VMEM),
            pl.BlockSpec(memory_space=pl.ANY),
        ],
        out_specs=pl.BlockSpec(memory_space=pltpu.VMEM),
        scratch_shapes=[
            pltpu.VMEM((N_DEV, k, blk), jnp.float32),
            pltpu.VMEM((m_per, k), jnp.bfloat16),
            pltpu.VMEM((N_DEV, m_per, blk), jnp.bfloat16),
            pltpu.SemaphoreType.DMA((N_DEV,)),
            pltpu.SemaphoreType.DMA((N_DEV,)),
            pltpu.SemaphoreType.DMA((N_DEV,)),
        ],
        compiler_params=pltpu.CompilerParams(collective_id=0),
    )(x, w_mat)
